# Initial kernel scaffold; baseline (speedup 1.0000x reference)
import jax
import jax.numpy as jnp
from jax.experimental import pallas as pl


def kernel(x, msg_W1, msg_b1, msg_W2, msg_b2, upd_W1, upd_b1, upd_W2, upd_b2):
    raise NotImplementedError("write your pallas kernel here")


# speedup vs baseline: 8.2814x; 8.2814x over previous
import functools

import jax
import jax.numpy as jnp
from jax import lax
from jax.experimental import pallas as pl
from jax.experimental.pallas import tpu as pltpu
from jax.experimental.pallas import tpu_sc as plsc

N = 4096
C = 256
K = 20

BLK = 256
NBLK = N // BLK


def _tc1_body(x_ref, w1a_ref, w1b_ref, b1_ref,
              idx_ref, w_ref, p_ref, b_ref, xn_ref):
    i = pl.program_id(0)

    @pl.when(i == 0)
    def _():
        xf = x_ref[...]
        nrm = jnp.maximum(
            jnp.sqrt(jnp.sum(xf * xf, axis=1, keepdims=True)), 1e-8)
        xn_ref[...] = xf / nrm

    xblk = x_ref[pl.ds(i * BLK, BLK), :]
    p_ref[...] = (jnp.dot(xblk, w1a_ref[...],
                          preferred_element_type=jnp.float32) + b1_ref[...])
    b_ref[...] = jnp.dot(xblk, w1b_ref[...],
                         preferred_element_type=jnp.float32)

    xnblk = xn_ref[pl.ds(i * BLK, BLK), :]
    sim = lax.dot_general(xnblk, xn_ref[...],
                          (((1,), (1,)), ((), ())),
                          preferred_element_type=jnp.float32)

    iota = lax.broadcasted_iota(jnp.int32, (BLK, N), 1)
    s = sim
    vals, idxs = [], []
    for _ in range(K):
        m = jnp.max(s, axis=1, keepdims=True)
        eq = s == m
        ix = jnp.min(jnp.where(eq, iota, N), axis=1, keepdims=True)
        vals.append(m)
        idxs.append(ix)
        s = jnp.where(iota == ix, -jnp.inf, s)

    v = jnp.concatenate(vals, axis=1)
    ix = jnp.concatenate(idxs, axis=1)
    e = jnp.exp(v - v[:, 0:1])
    wgt = e / jnp.sum(e, axis=1, keepdims=True)
    idx_ref[...] = ix
    w_ref[...] = wgt


def _run_tc1(x, w1a, w1b, b1):
    return pl.pallas_call(
        _tc1_body,
        grid=(NBLK,),
        in_specs=[
            pl.BlockSpec((N, C), lambda i: (0, 0)),
            pl.BlockSpec((C, C), lambda i: (0, 0)),
            pl.BlockSpec((C, C), lambda i: (0, 0)),
            pl.BlockSpec((1, C), lambda i: (0, 0)),
        ],
        out_specs=[
            pl.BlockSpec((BLK, K), lambda i: (i, 0)),
            pl.BlockSpec((BLK, K), lambda i: (i, 0)),
            pl.BlockSpec((BLK, C), lambda i: (i, 0)),
            pl.BlockSpec((BLK, C), lambda i: (i, 0)),
        ],
        out_shape=[
            jax.ShapeDtypeStruct((N, K), jnp.int32),
            jax.ShapeDtypeStruct((N, K), jnp.float32),
            jax.ShapeDtypeStruct((N, C), jnp.float32),
            jax.ShapeDtypeStruct((N, C), jnp.float32),
        ],
        scratch_shapes=[pltpu.VMEM((N, C), jnp.float32)],
    )(x, w1a, w1b, b1)


_NC = 2
_NS = 16
_NW = _NC * _NS
_NPW = N // _NW
_BN = 4
_NB = _NPW // _BN
_L = 16


def _sc_body(p_hbm, b_hbm, idx_hbm, wexp_hbm, out_hbm,
             idx_v, wexp_v, p_v, rows_v, acc_v, sem):
    cid = lax.axis_index("c")
    sid = lax.axis_index("s")
    wid = sid * _NC + cid

    def batch(bi, carry):
        nbase = wid * _NPW + bi * _BN
        ebase = nbase * K
        pltpu.sync_copy(idx_hbm.at[pl.ds(ebase, _BN * K)], idx_v)
        pltpu.sync_copy(wexp_hbm.at[pl.ds(ebase, _BN * K)], wexp_v)
        pltpu.sync_copy(p_hbm.at[pl.ds(nbase, _BN)], p_v)
        pltpu.async_copy(b_hbm.at[idx_v], rows_v, sem).wait()
        for n in range(_BN):
            e0 = n * K
            ws = [wexp_v[e0 + k] for k in range(K)]
            for cc in range(C // _L):
                pc = p_v[n, pl.ds(cc * _L, _L)]
                acc = jnp.zeros((_L,), jnp.float32)
                for k in range(K):
                    r = rows_v[e0 + k, pl.ds(cc * _L, _L)]
                    acc = acc + ws[k] * jnp.maximum(pc + r, 0.0)
                acc_v[n, pl.ds(cc * _L, _L)] = acc
        pltpu.sync_copy(acc_v, out_hbm.at[pl.ds(nbase, _BN)])
        return carry

    lax.fori_loop(0, _NB, batch, 0)


@functools.cache
def _sc_kernel_built():
    return functools.partial(
        pl.kernel,
        out_type=jax.ShapeDtypeStruct((N, C), jnp.float32),
        mesh=plsc.VectorSubcoreMesh(
            core_axis_name="c", subcore_axis_name="s",
            num_cores=_NC, num_subcores=_NS),
        scratch_types=[
            pltpu.VMEM((_BN * K,), jnp.int32),
            pltpu.VMEM((_BN * K, _L), jnp.float32),
            pltpu.VMEM((_BN, C), jnp.float32),
            pltpu.VMEM((_BN * K, C), jnp.float32),
            pltpu.VMEM((_BN, C), jnp.float32),
            pltpu.SemaphoreType.DMA,
        ],
    )(_sc_body)


def _sc_kernel(p, b, idx_flat, wexp):
    return _sc_kernel_built()(p, b, idx_flat, wexp)


_BLK2 = 1024


def _tc2_body(x_ref, hsum_ref, mW2_ref, mb2_ref,
              uW1_ref, ub1_ref, uW2_ref, ub2_ref, out_ref):
    hagg = (jnp.dot(hsum_ref[...], mW2_ref[...],
                    preferred_element_type=jnp.float32) + mb2_ref[...])
    xb = x_ref[...]
    u = (jnp.dot(xb, uW1_ref[0:C, :], preferred_element_type=jnp.float32)
         + jnp.dot(hagg, uW1_ref[C:2 * C, :],
                   preferred_element_type=jnp.float32)
         + ub1_ref[...])
    u = jnp.maximum(u, 0.0)
    out_ref[...] = (xb + jnp.dot(u, uW2_ref[...],
                                 preferred_element_type=jnp.float32)
                    + ub2_ref[...])


def _run_tc2(x, hsum, mW2, mb2, uW1, ub1, uW2, ub2):
    nb = N // _BLK2
    return pl.pallas_call(
        _tc2_body,
        grid=(nb,),
        in_specs=[
            pl.BlockSpec((_BLK2, C), lambda i: (i, 0)),
            pl.BlockSpec((_BLK2, C), lambda i: (i, 0)),
            pl.BlockSpec((C, C), lambda i: (0, 0)),
            pl.BlockSpec((1, C), lambda i: (0, 0)),
            pl.BlockSpec((2 * C, C), lambda i: (0, 0)),
            pl.BlockSpec((1, C), lambda i: (0, 0)),
            pl.BlockSpec((C, C), lambda i: (0, 0)),
            pl.BlockSpec((1, C), lambda i: (0, 0)),
        ],
        out_specs=pl.BlockSpec((_BLK2, C), lambda i: (i, 0)),
        out_shape=jax.ShapeDtypeStruct((N, C), jnp.float32),
    )(x, hsum, mW2, mb2, uW1, ub1, uW2, ub2)


def kernel(x, msg_W1, msg_b1, msg_W2, msg_b2, upd_W1, upd_b1, upd_W2, upd_b2):
    w1a = msg_W1[:C]
    w1b = msg_W1[C:]
    idx, wgt, p, b = _run_tc1(x, w1a, w1b, msg_b1.reshape(1, C))
    idx_flat = idx.reshape(-1)
    wexp = jnp.broadcast_to(wgt.reshape(-1, 1), (N * K, _L))
    hsum = _sc_kernel(p, b, idx_flat, wexp)
    return _run_tc2(x, hsum, msg_W2, msg_b2.reshape(1, C),
                    upd_W1, upd_b1.reshape(1, C),
                    upd_W2, upd_b2.reshape(1, C))


# speedup vs baseline: 10.3326x; 1.2477x over previous
import functools

import jax
import jax.numpy as jnp
from jax import lax
from jax.experimental import pallas as pl
from jax.experimental.pallas import tpu as pltpu
from jax.experimental.pallas import tpu_sc as plsc

N = 4096
C = 256
K = 20

BLK = 256
NBLK = N // BLK


def _tc1_body(x_ref, w1a_ref, w1b_ref, b1_ref,
              idx_ref, w_ref, p_ref, b_ref, xn_ref):
    i = pl.program_id(0)

    @pl.when(i == 0)
    def _():
        xf = x_ref[...]
        nrm = jnp.maximum(
            jnp.sqrt(jnp.sum(xf * xf, axis=1, keepdims=True)), 1e-8)
        xn_ref[...] = xf / nrm

    xblk = x_ref[pl.ds(i * BLK, BLK), :]
    p_ref[...] = (jnp.dot(xblk, w1a_ref[...],
                          preferred_element_type=jnp.float32) + b1_ref[...])
    b_ref[...] = jnp.dot(xblk, w1b_ref[...],
                         preferred_element_type=jnp.float32)

    xnblk = xn_ref[pl.ds(i * BLK, BLK), :]
    sim = lax.dot_general(xnblk, xn_ref[...],
                          (((1,), (1,)), ((), ())),
                          preferred_element_type=jnp.float32)

    iota = lax.broadcasted_iota(jnp.int32, (BLK, N), 1)
    bits = lax.bitcast_convert_type(sim, jnp.int32)
    skey = jnp.where(bits >= 0, bits, bits ^ jnp.int32(0x7FFFFFFF))
    kk = (skey & jnp.int32(~0xFFF)) | (jnp.int32(N - 1) - iota)

    picked = []
    for _ in range(K):
        m = jnp.max(kk, axis=1, keepdims=True)
        picked.append(m)
        kk = jnp.where(kk == m, jnp.int32(-2147483648), kk)

    kcat = jnp.concatenate(picked, axis=1)
    ix = jnp.int32(N - 1) - (kcat & jnp.int32(0xFFF))
    t = kcat & jnp.int32(~0xFFF)
    vbits = jnp.where(t >= 0, t, t ^ jnp.int32(0x7FFFFFFF))
    v = lax.bitcast_convert_type(vbits, jnp.float32)
    e = jnp.exp(v - v[:, 0:1])
    wgt = e / jnp.sum(e, axis=1, keepdims=True)
    idx_ref[...] = ix
    w_ref[...] = wgt


def _run_tc1(x, w1a, w1b, b1):
    return pl.pallas_call(
        _tc1_body,
        grid=(NBLK,),
        in_specs=[
            pl.BlockSpec((N, C), lambda i: (0, 0)),
            pl.BlockSpec((C, C), lambda i: (0, 0)),
            pl.BlockSpec((C, C), lambda i: (0, 0)),
            pl.BlockSpec((1, C), lambda i: (0, 0)),
        ],
        out_specs=[
            pl.BlockSpec((BLK, K), lambda i: (i, 0)),
            pl.BlockSpec((BLK, K), lambda i: (i, 0)),
            pl.BlockSpec((BLK, C), lambda i: (i, 0)),
            pl.BlockSpec((BLK, C), lambda i: (i, 0)),
        ],
        out_shape=[
            jax.ShapeDtypeStruct((N, K), jnp.int32),
            jax.ShapeDtypeStruct((N, K), jnp.float32),
            jax.ShapeDtypeStruct((N, C), jnp.float32),
            jax.ShapeDtypeStruct((N, C), jnp.float32),
        ],
        scratch_shapes=[pltpu.VMEM((N, C), jnp.float32)],
    )(x, w1a, w1b, b1)


_NC = 2
_NS = 16
_NW = _NC * _NS
_NPW = N // _NW
_BN = 4
_NB = _NPW // _BN
_L = 16


def _sc_body(p_hbm, b_hbm, idx_hbm, wexp_hbm, out_hbm,
             idx_v, wexp_v, p_v, rows_v, acc_v, sem):
    cid = lax.axis_index("c")
    sid = lax.axis_index("s")
    wid = sid * _NC + cid

    def batch(bi, carry):
        nbase = wid * _NPW + bi * _BN
        ebase = nbase * K
        pltpu.sync_copy(idx_hbm.at[pl.ds(ebase, _BN * K)], idx_v)
        pltpu.sync_copy(wexp_hbm.at[pl.ds(ebase, _BN * K)], wexp_v)
        pltpu.sync_copy(p_hbm.at[pl.ds(nbase, _BN)], p_v)
        pltpu.async_copy(b_hbm.at[idx_v], rows_v, sem).wait()
        for n in range(_BN):
            e0 = n * K
            ws = [wexp_v[e0 + k] for k in range(K)]
            for cc in range(C // _L):
                pc = p_v[n, pl.ds(cc * _L, _L)]
                acc = jnp.zeros((_L,), jnp.float32)
                for k in range(K):
                    r = rows_v[e0 + k, pl.ds(cc * _L, _L)]
                    acc = acc + ws[k] * jnp.maximum(pc + r, 0.0)
                acc_v[n, pl.ds(cc * _L, _L)] = acc
        pltpu.sync_copy(acc_v, out_hbm.at[pl.ds(nbase, _BN)])
        return carry

    lax.fori_loop(0, _NB, batch, 0)


@functools.cache
def _sc_kernel_built():
    return functools.partial(
        pl.kernel,
        out_type=jax.ShapeDtypeStruct((N, C), jnp.float32),
        mesh=plsc.VectorSubcoreMesh(
            core_axis_name="c", subcore_axis_name="s",
            num_cores=_NC, num_subcores=_NS),
        scratch_types=[
            pltpu.VMEM((_BN * K,), jnp.int32),
            pltpu.VMEM((_BN * K, _L), jnp.float32),
            pltpu.VMEM((_BN, C), jnp.float32),
            pltpu.VMEM((_BN * K, C), jnp.float32),
            pltpu.VMEM((_BN, C), jnp.float32),
            pltpu.SemaphoreType.DMA,
        ],
    )(_sc_body)


def _sc_kernel(p, b, idx_flat, wexp):
    return _sc_kernel_built()(p, b, idx_flat, wexp)


_BLK2 = 1024


def _tc2_body(x_ref, hsum_ref, mW2_ref, mb2_ref,
              uW1_ref, ub1_ref, uW2_ref, ub2_ref, out_ref):
    hagg = (jnp.dot(hsum_ref[...], mW2_ref[...],
                    preferred_element_type=jnp.float32) + mb2_ref[...])
    xb = x_ref[...]
    u = (jnp.dot(xb, uW1_ref[0:C, :], preferred_element_type=jnp.float32)
         + jnp.dot(hagg, uW1_ref[C:2 * C, :],
                   preferred_element_type=jnp.float32)
         + ub1_ref[...])
    u = jnp.maximum(u, 0.0)
    out_ref[...] = (xb + jnp.dot(u, uW2_ref[...],
                                 preferred_element_type=jnp.float32)
                    + ub2_ref[...])


def _run_tc2(x, hsum, mW2, mb2, uW1, ub1, uW2, ub2):
    nb = N // _BLK2
    return pl.pallas_call(
        _tc2_body,
        grid=(nb,),
        in_specs=[
            pl.BlockSpec((_BLK2, C), lambda i: (i, 0)),
            pl.BlockSpec((_BLK2, C), lambda i: (i, 0)),
            pl.BlockSpec((C, C), lambda i: (0, 0)),
            pl.BlockSpec((1, C), lambda i: (0, 0)),
            pl.BlockSpec((2 * C, C), lambda i: (0, 0)),
            pl.BlockSpec((1, C), lambda i: (0, 0)),
            pl.BlockSpec((C, C), lambda i: (0, 0)),
            pl.BlockSpec((1, C), lambda i: (0, 0)),
        ],
        out_specs=pl.BlockSpec((_BLK2, C), lambda i: (i, 0)),
        out_shape=jax.ShapeDtypeStruct((N, C), jnp.float32),
    )(x, hsum, mW2, mb2, uW1, ub1, uW2, ub2)


def kernel(x, msg_W1, msg_b1, msg_W2, msg_b2, upd_W1, upd_b1, upd_W2, upd_b2):
    w1a = msg_W1[:C]
    w1b = msg_W1[C:]
    idx, wgt, p, b = _run_tc1(x, w1a, w1b, msg_b1.reshape(1, C))
    idx_flat = idx.reshape(-1)
    wexp = jnp.broadcast_to(wgt.reshape(-1, 1), (N * K, _L))
    hsum = _sc_kernel(p, b, idx_flat, wexp)
    return _run_tc2(x, hsum, msg_W2, msg_b2.reshape(1, C),
                    upd_W1, upd_b1.reshape(1, C),
                    upd_W2, upd_b2.reshape(1, C))


# speedup vs baseline: 10.8992x; 1.0548x over previous
import functools

import jax
import jax.numpy as jnp
from jax import lax
from jax.experimental import pallas as pl
from jax.experimental.pallas import tpu as pltpu
from jax.experimental.pallas import tpu_sc as plsc

N = 4096
C = 256
K = 20

BLK = 256
NBLK = N // BLK


def _tc1_body(x_ref, w1a_ref, w1b_ref, b1_ref,
              idx_ref, w_ref, p_ref, b_ref, xn_ref):
    i = pl.program_id(0)

    @pl.when(i == 0)
    def _():
        xf = x_ref[...]
        nrm = jnp.maximum(
            jnp.sqrt(jnp.sum(xf * xf, axis=1, keepdims=True)), 1e-8)
        xn_ref[...] = xf / nrm

    xblk = x_ref[pl.ds(i * BLK, BLK), :]
    p_ref[...] = (jnp.dot(xblk, w1a_ref[...],
                          preferred_element_type=jnp.float32) + b1_ref[...])
    b_ref[...] = jnp.dot(xblk, w1b_ref[...],
                         preferred_element_type=jnp.float32)

    xnblk = xn_ref[pl.ds(i * BLK, BLK), :]
    sim = lax.dot_general(xnblk, xn_ref[...],
                          (((1,), (1,)), ((), ())),
                          preferred_element_type=jnp.float32)

    iota = lax.broadcasted_iota(jnp.int32, (BLK, N), 1)
    bits = lax.bitcast_convert_type(sim, jnp.int32)
    skey = jnp.where(bits >= 0, bits, bits ^ jnp.int32(0x7FFFFFFF))
    kk = (skey & jnp.int32(~0xFFF)) | (jnp.int32(N - 1) - iota)

    picked = []
    for _ in range(K):
        m = jnp.max(kk, axis=1, keepdims=True)
        picked.append(m)
        kk = jnp.where(kk == m, jnp.int32(-2147483648), kk)

    kcat = jnp.concatenate(picked, axis=1)
    ix = jnp.int32(N - 1) - (kcat & jnp.int32(0xFFF))
    t = kcat & jnp.int32(~0xFFF)
    vbits = jnp.where(t >= 0, t, t ^ jnp.int32(0x7FFFFFFF))
    v = lax.bitcast_convert_type(vbits, jnp.float32)
    e = jnp.exp(v - v[:, 0:1])
    wgt = e / jnp.sum(e, axis=1, keepdims=True)
    idx_ref[...] = ix
    w_ref[...] = wgt


def _run_tc1(x, w1a, w1b, b1):
    return pl.pallas_call(
        _tc1_body,
        grid=(NBLK,),
        in_specs=[
            pl.BlockSpec((N, C), lambda i: (0, 0)),
            pl.BlockSpec((C, C), lambda i: (0, 0)),
            pl.BlockSpec((C, C), lambda i: (0, 0)),
            pl.BlockSpec((1, C), lambda i: (0, 0)),
        ],
        out_specs=[
            pl.BlockSpec((BLK, K), lambda i: (i, 0)),
            pl.BlockSpec((BLK, K), lambda i: (i, 0)),
            pl.BlockSpec((BLK, C), lambda i: (i, 0)),
            pl.BlockSpec((BLK, C), lambda i: (i, 0)),
        ],
        out_shape=[
            jax.ShapeDtypeStruct((N, K), jnp.int32),
            jax.ShapeDtypeStruct((N, K), jnp.float32),
            jax.ShapeDtypeStruct((N, C), jnp.float32),
            jax.ShapeDtypeStruct((N, C), jnp.float32),
        ],
        scratch_shapes=[pltpu.VMEM((N, C), jnp.float32)],
    )(x, w1a, w1b, b1)


_NC = 2
_NS = 16
_NW = _NC * _NS
_NPW = N // _NW
_BN = 4
_NB = _NPW // _BN
_L = 16


def _sc_body(p_hbm, b_hbm, idx_hbm, wexp_hbm, out_hbm,
             idx0, wexp0, p0, rows0, acc0,
             idx1, wexp1, p1, rows1, acc1,
             semin0, semg0, semout0, semin1, semg1, semout1):
    cid = lax.axis_index("c")
    sid = lax.axis_index("s")
    wid = sid * _NC + cid
    node0 = wid * _NPW

    bufs = ((idx0, wexp0, p0, rows0, acc0, semin0, semg0, semout0),
            (idx1, wexp1, p1, rows1, acc1, semin1, semg1, semout1))

    def fire_in(bi, q):
        idx_v, wexp_v, p_v, _, _, semin, _, _ = bufs[q]
        nb = node0 + bi * _BN
        eb = nb * K
        pltpu.async_copy(idx_hbm.at[pl.ds(eb, _BN * K)], idx_v, semin)
        pltpu.async_copy(wexp_hbm.at[pl.ds(eb, _BN * K)], wexp_v, semin)
        pltpu.async_copy(p_hbm.at[pl.ds(nb, _BN)], p_v, semin)

    def wait_in(q):
        idx_v, wexp_v, p_v, _, _, semin, _, _ = bufs[q]
        pltpu.make_async_copy(idx_hbm.at[pl.ds(0, _BN * K)], idx_v,
                              semin).wait()
        pltpu.make_async_copy(wexp_hbm.at[pl.ds(0, _BN * K)], wexp_v,
                              semin).wait()
        pltpu.make_async_copy(p_hbm.at[pl.ds(0, _BN)], p_v, semin).wait()

    def fire_gather(q):
        idx_v, _, _, rows_v, _, _, semg, _ = bufs[q]
        pltpu.async_copy(b_hbm.at[idx_v], rows_v, semg)

    def wait_gather(q):
        idx_v, _, _, rows_v, _, _, semg, _ = bufs[q]
        pltpu.make_async_copy(b_hbm.at[idx_v], rows_v, semg).wait()

    def fire_out(bi, q):
        acc_v, semout = bufs[q][4], bufs[q][7]
        nb = node0 + bi * _BN
        pltpu.async_copy(acc_v, out_hbm.at[pl.ds(nb, _BN)], semout)

    def wait_out(q):
        acc_v, semout = bufs[q][4], bufs[q][7]
        pltpu.make_async_copy(acc_v, out_hbm.at[pl.ds(0, _BN)],
                              semout).wait()

    def compute(q):
        _, wexp_v, p_v, rows_v, acc_v, _, _, _ = bufs[q]
        for n in range(_BN):
            e0 = n * K
            ws = [wexp_v[e0 + k] for k in range(K)]
            for cc in range(C // _L):
                pc = p_v[n, pl.ds(cc * _L, _L)]
                acc = jnp.zeros((_L,), jnp.float32)
                for k in range(K):
                    r = rows_v[e0 + k, pl.ds(cc * _L, _L)]
                    acc = acc + ws[k] * jnp.maximum(pc + r, 0.0)
                acc_v[n, pl.ds(cc * _L, _L)] = acc

    fire_in(0, 0)
    wait_in(0)
    fire_gather(0)
    fire_in(1, 1)

    def step(t, carry):
        for q in range(2):
            bi = 2 * t + q
            wait_gather(q)

            @pl.when(bi >= 2)
            def _():
                wait_out(q)

            compute(q)
            fire_out(bi, q)

            @pl.when(bi + 2 < _NB)
            def _():
                fire_in(bi + 2, q)

            @pl.when(bi + 1 < _NB)
            def _():
                wait_in(1 - q)
                fire_gather(1 - q)
        return carry

    lax.fori_loop(0, _NB // 2, step, 0)
    wait_out(0)
    wait_out(1)


@functools.cache
def _sc_kernel_built():
    return functools.partial(
        pl.kernel,
        out_type=jax.ShapeDtypeStruct((N, C), jnp.float32),
        mesh=plsc.VectorSubcoreMesh(
            core_axis_name="c", subcore_axis_name="s",
            num_cores=_NC, num_subcores=_NS),
        scratch_types=(
            [pltpu.VMEM((_BN * K,), jnp.int32),
             pltpu.VMEM((_BN * K, _L), jnp.float32),
             pltpu.VMEM((_BN, C), jnp.float32),
             pltpu.VMEM((_BN * K, C), jnp.float32),
             pltpu.VMEM((_BN, C), jnp.float32)] * 2
            + [pltpu.SemaphoreType.DMA] * 6
        ),
    )(_sc_body)


def _sc_kernel(p, b, idx_flat, wexp):
    return _sc_kernel_built()(p, b, idx_flat, wexp)


_BLK2 = 1024


def _tc2_body(x_ref, hsum_ref, mW2_ref, mb2_ref,
              uW1_ref, ub1_ref, uW2_ref, ub2_ref, out_ref):
    hagg = (jnp.dot(hsum_ref[...], mW2_ref[...],
                    preferred_element_type=jnp.float32) + mb2_ref[...])
    xb = x_ref[...]
    u = (jnp.dot(xb, uW1_ref[0:C, :], preferred_element_type=jnp.float32)
         + jnp.dot(hagg, uW1_ref[C:2 * C, :],
                   preferred_element_type=jnp.float32)
         + ub1_ref[...])
    u = jnp.maximum(u, 0.0)
    out_ref[...] = (xb + jnp.dot(u, uW2_ref[...],
                                 preferred_element_type=jnp.float32)
                    + ub2_ref[...])


def _run_tc2(x, hsum, mW2, mb2, uW1, ub1, uW2, ub2):
    nb = N // _BLK2
    return pl.pallas_call(
        _tc2_body,
        grid=(nb,),
        in_specs=[
            pl.BlockSpec((_BLK2, C), lambda i: (i, 0)),
            pl.BlockSpec((_BLK2, C), lambda i: (i, 0)),
            pl.BlockSpec((C, C), lambda i: (0, 0)),
            pl.BlockSpec((1, C), lambda i: (0, 0)),
            pl.BlockSpec((2 * C, C), lambda i: (0, 0)),
            pl.BlockSpec((1, C), lambda i: (0, 0)),
            pl.BlockSpec((C, C), lambda i: (0, 0)),
            pl.BlockSpec((1, C), lambda i: (0, 0)),
        ],
        out_specs=pl.BlockSpec((_BLK2, C), lambda i: (i, 0)),
        out_shape=jax.ShapeDtypeStruct((N, C), jnp.float32),
    )(x, hsum, mW2, mb2, uW1, ub1, uW2, ub2)


def kernel(x, msg_W1, msg_b1, msg_W2, msg_b2, upd_W1, upd_b1, upd_W2, upd_b2):
    w1a = msg_W1[:C]
    w1b = msg_W1[C:]
    idx, wgt, p, b = _run_tc1(x, w1a, w1b, msg_b1.reshape(1, C))
    idx_flat = idx.reshape(-1)
    wexp = jnp.broadcast_to(wgt.reshape(-1, 1), (N * K, _L))
    hsum = _sc_kernel(p, b, idx_flat, wexp)
    return _run_tc2(x, hsum, msg_W2, msg_b2.reshape(1, C),
                    upd_W1, upd_b1.reshape(1, C),
                    upd_W2, upd_b2.reshape(1, C))


# speedup vs baseline: 11.4372x; 1.0494x over previous
import functools

import jax
import jax.numpy as jnp
from jax import lax
from jax.experimental import pallas as pl
from jax.experimental.pallas import tpu as pltpu
from jax.experimental.pallas import tpu_sc as plsc

N = 4096
C = 256
K = 20

BLK = 256
NBLK = N // BLK


def _tc1_body(x_ref, w1a_ref, w1b_ref, b1_ref,
              idx_ref, w_ref, p_ref, b_ref, xn_ref):
    i = pl.program_id(0)

    @pl.when(i == 0)
    def _():
        xf = x_ref[...]
        nrm = jnp.maximum(
            jnp.sqrt(jnp.sum(xf * xf, axis=1, keepdims=True)), 1e-8)
        xn_ref[...] = xf / nrm

    xblk = x_ref[pl.ds(i * BLK, BLK), :]
    p_ref[...] = (jnp.dot(xblk, w1a_ref[...],
                          preferred_element_type=jnp.float32) + b1_ref[...])
    b_ref[...] = jnp.dot(xblk, w1b_ref[...],
                         preferred_element_type=jnp.float32)

    xnblk = xn_ref[pl.ds(i * BLK, BLK), :]
    sim = lax.dot_general(xnblk, xn_ref[...],
                          (((1,), (1,)), ((), ())),
                          preferred_element_type=jnp.float32)

    iota = lax.broadcasted_iota(jnp.int32, (BLK, N), 1)
    bits = lax.bitcast_convert_type(sim, jnp.int32)
    skey = jnp.where(bits >= 0, bits, bits ^ jnp.int32(0x7FFFFFFF))
    kk = (skey & jnp.int32(~0xFFF)) | (jnp.int32(N - 1) - iota)

    picked = []
    for _ in range(K):
        m = jnp.max(kk, axis=1, keepdims=True)
        picked.append(m)
        kk = jnp.where(kk == m, jnp.int32(-2147483648), kk)

    kcat = jnp.concatenate(picked, axis=1)
    ix = jnp.int32(N - 1) - (kcat & jnp.int32(0xFFF))
    t = kcat & jnp.int32(~0xFFF)
    vbits = jnp.where(t >= 0, t, t ^ jnp.int32(0x7FFFFFFF))
    v = lax.bitcast_convert_type(vbits, jnp.float32)
    e = jnp.exp(v - v[:, 0:1])
    wgt = e / jnp.sum(e, axis=1, keepdims=True)
    idx_ref[...] = ix
    w_ref[...] = wgt


def _run_tc1(x, w1a, w1b, b1):
    return pl.pallas_call(
        _tc1_body,
        grid=(NBLK,),
        in_specs=[
            pl.BlockSpec((N, C), lambda i: (0, 0)),
            pl.BlockSpec((C, C), lambda i: (0, 0)),
            pl.BlockSpec((C, C), lambda i: (0, 0)),
            pl.BlockSpec((1, C), lambda i: (0, 0)),
        ],
        out_specs=[
            pl.BlockSpec((BLK, K), lambda i: (i, 0)),
            pl.BlockSpec((BLK, K), lambda i: (i, 0)),
            pl.BlockSpec((BLK, C), lambda i: (i, 0)),
            pl.BlockSpec((BLK, C), lambda i: (i, 0)),
        ],
        out_shape=[
            jax.ShapeDtypeStruct((N, K), jnp.int32),
            jax.ShapeDtypeStruct((N, K), jnp.float32),
            jax.ShapeDtypeStruct((N, C), jnp.float32),
            jax.ShapeDtypeStruct((N, C), jnp.float32),
        ],
        scratch_shapes=[pltpu.VMEM((N, C), jnp.float32)],
    )(x, w1a, w1b, b1)


_NC = 2
_NS = 16
_NW = _NC * _NS
_NPW = N // _NW
_BN = 4
_NB = _NPW // _BN
_L = 16


def _sc_body(p_hbm, b_hbm, idx_hbm, wexp_hbm, out_hbm,
             idx0, wexp0, p0, rows0, acc0,
             idx1, wexp1, p1, rows1, acc1,
             semin0, semg0, semout0, semin1, semg1, semout1):
    cid = lax.axis_index("c")
    sid = lax.axis_index("s")
    wid = sid * _NC + cid
    node0 = wid * _NPW

    bufs = ((idx0, wexp0, p0, rows0, acc0, semin0, semg0, semout0),
            (idx1, wexp1, p1, rows1, acc1, semin1, semg1, semout1))

    def fire_in(bi, q):
        idx_v, wexp_v, p_v, _, _, semin, _, _ = bufs[q]
        nb = node0 + bi * _BN
        eb = nb * K
        pltpu.async_copy(idx_hbm.at[pl.ds(eb, _BN * K)], idx_v, semin)
        pltpu.async_copy(wexp_hbm.at[pl.ds(eb, _BN * K)], wexp_v, semin)
        pltpu.async_copy(p_hbm.at[pl.ds(nb, _BN)], p_v, semin)

    def wait_in(q):
        idx_v, wexp_v, p_v, _, _, semin, _, _ = bufs[q]
        pltpu.make_async_copy(idx_hbm.at[pl.ds(0, _BN * K)], idx_v,
                              semin).wait()
        pltpu.make_async_copy(wexp_hbm.at[pl.ds(0, _BN * K)], wexp_v,
                              semin).wait()
        pltpu.make_async_copy(p_hbm.at[pl.ds(0, _BN)], p_v, semin).wait()

    def fire_gather(q):
        idx_v, _, _, rows_v, _, _, semg, _ = bufs[q]
        pltpu.async_copy(b_hbm.at[idx_v], rows_v, semg)

    def wait_gather(q):
        idx_v, _, _, rows_v, _, _, semg, _ = bufs[q]
        pltpu.make_async_copy(b_hbm.at[idx_v], rows_v, semg).wait()

    def fire_out(bi, q):
        acc_v, semout = bufs[q][4], bufs[q][7]
        nb = node0 + bi * _BN
        pltpu.async_copy(acc_v, out_hbm.at[pl.ds(nb, _BN)], semout)

    def wait_out(q):
        acc_v, semout = bufs[q][4], bufs[q][7]
        pltpu.make_async_copy(acc_v, out_hbm.at[pl.ds(0, _BN)],
                              semout).wait()

    def compute(q):
        _, wexp_v, p_v, rows_v, acc_v, _, _, _ = bufs[q]
        for n in range(_BN):
            e0 = n * K
            ws = [wexp_v[e0 + k] for k in range(K)]
            for cc in range(C // _L):
                pc = p_v[n, pl.ds(cc * _L, _L)]
                acc = jnp.zeros((_L,), jnp.float32)
                for k in range(K):
                    r = rows_v[e0 + k, pl.ds(cc * _L, _L)]
                    acc = acc + ws[k] * jnp.maximum(pc + r, 0.0)
                acc_v[n, pl.ds(cc * _L, _L)] = acc

    fire_in(0, 0)
    wait_in(0)
    fire_gather(0)
    fire_in(1, 1)

    def step(t, carry):
        for q in range(2):
            bi = 2 * t + q

            @pl.when(bi + 1 < _NB)
            def _():
                wait_in(1 - q)
                fire_gather(1 - q)

            wait_gather(q)

            @pl.when(bi >= 2)
            def _():
                wait_out(q)

            compute(q)
            fire_out(bi, q)

            @pl.when(bi + 2 < _NB)
            def _():
                fire_in(bi + 2, q)
        return carry

    lax.fori_loop(0, _NB // 2, step, 0)
    wait_out(0)
    wait_out(1)


@functools.cache
def _sc_kernel_built():
    return functools.partial(
        pl.kernel,
        out_type=jax.ShapeDtypeStruct((N, C), jnp.float32),
        mesh=plsc.VectorSubcoreMesh(
            core_axis_name="c", subcore_axis_name="s",
            num_cores=_NC, num_subcores=_NS),
        scratch_types=(
            [pltpu.VMEM((_BN * K,), jnp.int32),
             pltpu.VMEM((_BN * K, _L), jnp.float32),
             pltpu.VMEM((_BN, C), jnp.float32),
             pltpu.VMEM((_BN * K, C), jnp.float32),
             pltpu.VMEM((_BN, C), jnp.float32)] * 2
            + [pltpu.SemaphoreType.DMA] * 6
        ),
    )(_sc_body)


def _sc_kernel(p, b, idx_flat, wexp):
    return _sc_kernel_built()(p, b, idx_flat, wexp)


_BLK2 = 1024


def _tc2_body(x_ref, hsum_ref, mW2_ref, mb2_ref,
              uW1_ref, ub1_ref, uW2_ref, ub2_ref, out_ref):
    hagg = (jnp.dot(hsum_ref[...], mW2_ref[...],
                    preferred_element_type=jnp.float32) + mb2_ref[...])
    xb = x_ref[...]
    u = (jnp.dot(xb, uW1_ref[0:C, :], preferred_element_type=jnp.float32)
         + jnp.dot(hagg, uW1_ref[C:2 * C, :],
                   preferred_element_type=jnp.float32)
         + ub1_ref[...])
    u = jnp.maximum(u, 0.0)
    out_ref[...] = (xb + jnp.dot(u, uW2_ref[...],
                                 preferred_element_type=jnp.float32)
                    + ub2_ref[...])


def _run_tc2(x, hsum, mW2, mb2, uW1, ub1, uW2, ub2):
    nb = N // _BLK2
    return pl.pallas_call(
        _tc2_body,
        grid=(nb,),
        in_specs=[
            pl.BlockSpec((_BLK2, C), lambda i: (i, 0)),
            pl.BlockSpec((_BLK2, C), lambda i: (i, 0)),
            pl.BlockSpec((C, C), lambda i: (0, 0)),
            pl.BlockSpec((1, C), lambda i: (0, 0)),
            pl.BlockSpec((2 * C, C), lambda i: (0, 0)),
            pl.BlockSpec((1, C), lambda i: (0, 0)),
            pl.BlockSpec((C, C), lambda i: (0, 0)),
            pl.BlockSpec((1, C), lambda i: (0, 0)),
        ],
        out_specs=pl.BlockSpec((_BLK2, C), lambda i: (i, 0)),
        out_shape=jax.ShapeDtypeStruct((N, C), jnp.float32),
    )(x, hsum, mW2, mb2, uW1, ub1, uW2, ub2)


def kernel(x, msg_W1, msg_b1, msg_W2, msg_b2, upd_W1, upd_b1, upd_W2, upd_b2):
    w1a = msg_W1[:C]
    w1b = msg_W1[C:]
    idx, wgt, p, b = _run_tc1(x, w1a, w1b, msg_b1.reshape(1, C))
    idx_flat = idx.reshape(-1)
    wexp = jnp.broadcast_to(wgt.reshape(-1, 1), (N * K, _L))
    hsum = _sc_kernel(p, b, idx_flat, wexp)
    return _run_tc2(x, hsum, msg_W2, msg_b2.reshape(1, C),
                    upd_W1, upd_b1.reshape(1, C),
                    upd_W2, upd_b2.reshape(1, C))


# speedup vs baseline: 13.0453x; 1.1406x over previous
import functools

import jax
import jax.numpy as jnp
from jax import lax
from jax.experimental import pallas as pl
from jax.experimental.pallas import tpu as pltpu
from jax.experimental.pallas import tpu_sc as plsc

N = 4096
C = 256
K = 20

BLK = 256
NBLK = N // BLK


def _tc1_body(x_ref, w1a_ref, w1b_ref, b1_ref,
              idx_ref, w_ref, p_ref, b_ref, xn_ref):
    i = pl.program_id(0)

    @pl.when(i == 0)
    def _():
        xf = x_ref[...]
        nrm = jnp.maximum(
            jnp.sqrt(jnp.sum(xf * xf, axis=1, keepdims=True)), 1e-8)
        xn_ref[...] = xf / nrm

    xblk = x_ref[pl.ds(i * BLK, BLK), :]
    p_ref[...] = (jnp.dot(xblk, w1a_ref[...],
                          preferred_element_type=jnp.float32) + b1_ref[...])
    b_ref[...] = jnp.dot(xblk, w1b_ref[...],
                         preferred_element_type=jnp.float32)

    xnblk = xn_ref[pl.ds(i * BLK, BLK), :]
    sim = lax.dot_general(xnblk, xn_ref[...],
                          (((1,), (1,)), ((), ())),
                          preferred_element_type=jnp.float32)

    iota = lax.broadcasted_iota(jnp.int32, (BLK, N), 1)
    bits = lax.bitcast_convert_type(sim, jnp.int32)
    skey = jnp.where(bits >= 0, bits, bits ^ jnp.int32(0x7FFFFFFF))
    kk = (skey & jnp.int32(~0xFFF)) | (jnp.int32(N - 1) - iota)

    picked = []
    for _ in range(K):
        m = jnp.max(kk, axis=1, keepdims=True)
        picked.append(m)
        kk = jnp.where(kk == m, jnp.int32(-2147483648), kk)

    kcat = jnp.concatenate(picked, axis=1)
    ix = jnp.int32(N - 1) - (kcat & jnp.int32(0xFFF))
    t = kcat & jnp.int32(~0xFFF)
    vbits = jnp.where(t >= 0, t, t ^ jnp.int32(0x7FFFFFFF))
    v = lax.bitcast_convert_type(vbits, jnp.float32)
    e = jnp.exp(v - v[:, 0:1])
    wgt = e / jnp.sum(e, axis=1, keepdims=True)
    idx_ref[...] = ix
    w_ref[...] = wgt


def _run_tc1(x, w1a, w1b, b1):
    return pl.pallas_call(
        _tc1_body,
        grid=(NBLK,),
        in_specs=[
            pl.BlockSpec((N, C), lambda i: (0, 0)),
            pl.BlockSpec((C, C), lambda i: (0, 0)),
            pl.BlockSpec((C, C), lambda i: (0, 0)),
            pl.BlockSpec((1, C), lambda i: (0, 0)),
        ],
        out_specs=[
            pl.BlockSpec((BLK, K), lambda i: (i, 0)),
            pl.BlockSpec((BLK, K), lambda i: (i, 0)),
            pl.BlockSpec((BLK, C), lambda i: (i, 0)),
            pl.BlockSpec((BLK, C), lambda i: (i, 0)),
        ],
        out_shape=[
            jax.ShapeDtypeStruct((N, K), jnp.int32),
            jax.ShapeDtypeStruct((N, K), jnp.float32),
            jax.ShapeDtypeStruct((N, C), jnp.float32),
            jax.ShapeDtypeStruct((N, C), jnp.float32),
        ],
        scratch_shapes=[pltpu.VMEM((N, C), jnp.float32)],
    )(x, w1a, w1b, b1)


_NC = 2
_NS = 16
_NW = _NC * _NS
_NPW = N // _NW
_BN = 4
_NB = _NPW // _BN
_L = 16


def _sc_body(p_hbm, b_hbm, idx_hbm, wexp_hbm, out_hbm,
             idx0, idx1, idx2, idx3,
             wexp0, wexp1, wexp2, wexp3,
             p0, p1, p2, p3,
             rows0, rows1, acc0, acc1,
             semin0, semin1, semin2, semin3,
             semg0, semg1, semout0, semout1):
    cid = lax.axis_index("c")
    sid = lax.axis_index("s")
    wid = sid * _NC + cid
    node0 = wid * _NPW

    idxb = (idx0, idx1, idx2, idx3)
    wexpb = (wexp0, wexp1, wexp2, wexp3)
    pb = (p0, p1, p2, p3)
    seminb = (semin0, semin1, semin2, semin3)
    rowsb = (rows0, rows1)
    semgb = (semg0, semg1)
    accb = (acc0, acc1)
    semoutb = (semout0, semout1)

    def fire_in(bi, r):
        nb = node0 + bi * _BN
        eb = nb * K
        pltpu.async_copy(idx_hbm.at[pl.ds(eb, _BN * K)], idxb[r], seminb[r])
        pltpu.async_copy(wexp_hbm.at[pl.ds(eb, _BN * K)], wexpb[r],
                         seminb[r])
        pltpu.async_copy(p_hbm.at[pl.ds(nb, _BN)], pb[r], seminb[r])

    def wait_in(r):
        pltpu.make_async_copy(idx_hbm.at[pl.ds(0, _BN * K)], idxb[r],
                              seminb[r]).wait()
        pltpu.make_async_copy(wexp_hbm.at[pl.ds(0, _BN * K)], wexpb[r],
                              seminb[r]).wait()
        pltpu.make_async_copy(p_hbm.at[pl.ds(0, _BN)], pb[r],
                              seminb[r]).wait()

    def fire_gather(r, q):
        pltpu.async_copy(b_hbm.at[idxb[r]], rowsb[q], semgb[q])

    def wait_gather(r, q):
        pltpu.make_async_copy(b_hbm.at[idxb[r]], rowsb[q],
                              semgb[q]).wait()

    def fire_out(bi, q):
        nb = node0 + bi * _BN
        pltpu.async_copy(accb[q], out_hbm.at[pl.ds(nb, _BN)], semoutb[q])

    def wait_out(q):
        pltpu.make_async_copy(accb[q], out_hbm.at[pl.ds(0, _BN)],
                              semoutb[q]).wait()

    def compute(r, q):
        wexp_v, p_v, rows_v, acc_v = wexpb[r], pb[r], rowsb[q], accb[q]

        def node(n, carry):
            e0 = n * K
            ws = [wexp_v[e0 + k] for k in range(K)]
            for cc in range(C // _L):
                pc = p_v[n, pl.ds(cc * _L, _L)]
                acc = jnp.zeros((_L,), jnp.float32)
                for k in range(K):
                    rr = rows_v[e0 + k, pl.ds(cc * _L, _L)]
                    acc = acc + ws[k] * jnp.maximum(pc + rr, 0.0)
                acc_v[n, pl.ds(cc * _L, _L)] = acc
            return carry

        lax.fori_loop(0, _BN, node, 0)

    for r in range(4):
        fire_in(r, r)
    wait_in(0)
    fire_gather(0, 0)

    def step(t, carry):
        for u in range(4):
            bi = 4 * t + u
            q = u % 2
            r = u

            @pl.when(bi + 1 < _NB)
            def _():
                wait_in((r + 1) % 4)
                fire_gather((r + 1) % 4, 1 - q)

            wait_gather(r, q)

            @pl.when(bi >= 2)
            def _():
                wait_out(q)

            compute(r, q)
            fire_out(bi, q)

            @pl.when(bi + 4 < _NB)
            def _():
                fire_in(bi + 4, r)
        return carry

    lax.fori_loop(0, _NB // 4, step, 0)
    wait_out(0)
    wait_out(1)


@functools.cache
def _sc_kernel_built():
    return functools.partial(
        pl.kernel,
        out_type=jax.ShapeDtypeStruct((N, C), jnp.float32),
        mesh=plsc.VectorSubcoreMesh(
            core_axis_name="c", subcore_axis_name="s",
            num_cores=_NC, num_subcores=_NS),
        scratch_types=(
            [pltpu.VMEM((_BN * K,), jnp.int32)] * 4
            + [pltpu.VMEM((_BN * K, _L), jnp.float32)] * 4
            + [pltpu.VMEM((_BN, C), jnp.float32)] * 4
            + [pltpu.VMEM((_BN * K, C), jnp.float32)] * 2
            + [pltpu.VMEM((_BN, C), jnp.float32)] * 2
            + [pltpu.SemaphoreType.DMA] * 8
        ),
    )(_sc_body)


def _sc_kernel(p, b, idx_flat, wexp):
    return _sc_kernel_built()(p, b, idx_flat, wexp)


_BLK2 = 1024


def _tc2_body(x_ref, hsum_ref, mW2_ref, mb2_ref,
              uW1_ref, ub1_ref, uW2_ref, ub2_ref, out_ref):
    hagg = (jnp.dot(hsum_ref[...], mW2_ref[...],
                    preferred_element_type=jnp.float32) + mb2_ref[...])
    xb = x_ref[...]
    u = (jnp.dot(xb, uW1_ref[0:C, :], preferred_element_type=jnp.float32)
         + jnp.dot(hagg, uW1_ref[C:2 * C, :],
                   preferred_element_type=jnp.float32)
         + ub1_ref[...])
    u = jnp.maximum(u, 0.0)
    out_ref[...] = (xb + jnp.dot(u, uW2_ref[...],
                                 preferred_element_type=jnp.float32)
                    + ub2_ref[...])


def _run_tc2(x, hsum, mW2, mb2, uW1, ub1, uW2, ub2):
    nb = N // _BLK2
    return pl.pallas_call(
        _tc2_body,
        grid=(nb,),
        in_specs=[
            pl.BlockSpec((_BLK2, C), lambda i: (i, 0)),
            pl.BlockSpec((_BLK2, C), lambda i: (i, 0)),
            pl.BlockSpec((C, C), lambda i: (0, 0)),
            pl.BlockSpec((1, C), lambda i: (0, 0)),
            pl.BlockSpec((2 * C, C), lambda i: (0, 0)),
            pl.BlockSpec((1, C), lambda i: (0, 0)),
            pl.BlockSpec((C, C), lambda i: (0, 0)),
            pl.BlockSpec((1, C), lambda i: (0, 0)),
        ],
        out_specs=pl.BlockSpec((_BLK2, C), lambda i: (i, 0)),
        out_shape=jax.ShapeDtypeStruct((N, C), jnp.float32),
    )(x, hsum, mW2, mb2, uW1, ub1, uW2, ub2)


def kernel(x, msg_W1, msg_b1, msg_W2, msg_b2, upd_W1, upd_b1, upd_W2, upd_b2):
    w1a = msg_W1[:C]
    w1b = msg_W1[C:]
    idx, wgt, p, b = _run_tc1(x, w1a, w1b, msg_b1.reshape(1, C))
    idx_flat = idx.reshape(-1)
    wexp = jnp.broadcast_to(wgt.reshape(-1, 1), (N * K, _L))
    hsum = _sc_kernel(p, b, idx_flat, wexp)
    return _run_tc2(x, hsum, msg_W2, msg_b2.reshape(1, C),
                    upd_W1, upd_b1.reshape(1, C),
                    upd_W2, upd_b2.reshape(1, C))


# speedup vs baseline: 13.5132x; 1.0359x over previous
import functools

import jax
import jax.numpy as jnp
from jax import lax
from jax.experimental import pallas as pl
from jax.experimental.pallas import tpu as pltpu
from jax.experimental.pallas import tpu_sc as plsc

N = 4096
C = 256
K = 20

BLK = 256
NBLK = N // BLK


def _tc1_body(x_ref, w1a_ref, w1b_ref, b1_ref,
              idx_ref, w_ref, p_ref, b_ref, xn_ref):
    i = pl.program_id(0)

    @pl.when(i == 0)
    def _():
        xf = x_ref[...]
        nrm = jnp.maximum(
            jnp.sqrt(jnp.sum(xf * xf, axis=1, keepdims=True)), 1e-8)
        xn_ref[...] = xf / nrm

    xblk = x_ref[pl.ds(i * BLK, BLK), :]
    p_ref[...] = (jnp.dot(xblk, w1a_ref[...],
                          preferred_element_type=jnp.float32) + b1_ref[...])
    b_ref[...] = jnp.dot(xblk, w1b_ref[...],
                         preferred_element_type=jnp.float32
                         ).astype(jnp.bfloat16)

    xnblk = xn_ref[pl.ds(i * BLK, BLK), :]
    sim = lax.dot_general(xnblk, xn_ref[...],
                          (((1,), (1,)), ((), ())),
                          preferred_element_type=jnp.float32)

    iota = lax.broadcasted_iota(jnp.int32, (BLK, N), 1)
    bits = lax.bitcast_convert_type(sim, jnp.int32)
    skey = jnp.where(bits >= 0, bits, bits ^ jnp.int32(0x7FFFFFFF))
    kk = (skey & jnp.int32(~0xFFF)) | (jnp.int32(N - 1) - iota)

    picked = []
    for _ in range(K):
        m = jnp.max(kk, axis=1, keepdims=True)
        picked.append(m)
        kk = jnp.where(kk == m, jnp.int32(-2147483648), kk)

    kcat = jnp.concatenate(picked, axis=1)
    ix = jnp.int32(N - 1) - (kcat & jnp.int32(0xFFF))
    t = kcat & jnp.int32(~0xFFF)
    vbits = jnp.where(t >= 0, t, t ^ jnp.int32(0x7FFFFFFF))
    v = lax.bitcast_convert_type(vbits, jnp.float32)
    e = jnp.exp(v - v[:, 0:1])
    wgt = e / jnp.sum(e, axis=1, keepdims=True)
    idx_ref[...] = ix
    w_ref[...] = wgt


def _run_tc1(x, w1a, w1b, b1):
    return pl.pallas_call(
        _tc1_body,
        grid=(NBLK,),
        in_specs=[
            pl.BlockSpec((N, C), lambda i: (0, 0)),
            pl.BlockSpec((C, C), lambda i: (0, 0)),
            pl.BlockSpec((C, C), lambda i: (0, 0)),
            pl.BlockSpec((1, C), lambda i: (0, 0)),
        ],
        out_specs=[
            pl.BlockSpec((BLK, K), lambda i: (i, 0)),
            pl.BlockSpec((BLK, K), lambda i: (i, 0)),
            pl.BlockSpec((BLK, C), lambda i: (i, 0)),
            pl.BlockSpec((BLK, C), lambda i: (i, 0)),
        ],
        out_shape=[
            jax.ShapeDtypeStruct((N, K), jnp.int32),
            jax.ShapeDtypeStruct((N, K), jnp.float32),
            jax.ShapeDtypeStruct((N, C), jnp.float32),
            jax.ShapeDtypeStruct((N, C), jnp.bfloat16),
        ],
        scratch_shapes=[pltpu.VMEM((N, C), jnp.float32)],
    )(x, w1a, w1b, b1)


_NC = 2
_NS = 16
_NW = _NC * _NS
_NPW = N // _NW
_BN = 4
_NB = _NPW // _BN
_L = 32
_LW = _L // 2
_CW = C // 2


def _sc_body(p_hbm, b_hbm, idx_hbm, wexp_hbm, out_hbm,
             idx0, idx1, idx2, idx3,
             wexp0, wexp1, wexp2, wexp3,
             p0, p1, p2, p3,
             rows0, rows1, acc0, acc1,
             semin0, semin1, semin2, semin3,
             semg0, semg1, semout0, semout1):
    cid = lax.axis_index("c")
    sid = lax.axis_index("s")
    wid = sid * _NC + cid
    node0 = wid * _NPW

    idxb = (idx0, idx1, idx2, idx3)
    wexpb = (wexp0, wexp1, wexp2, wexp3)
    pb = (p0, p1, p2, p3)
    seminb = (semin0, semin1, semin2, semin3)
    rowsb = (rows0, rows1)
    semgb = (semg0, semg1)
    accb = (acc0, acc1)
    semoutb = (semout0, semout1)

    def fire_in(bi, r):
        nb = node0 + bi * _BN
        eb = nb * K
        pltpu.async_copy(idx_hbm.at[pl.ds(eb, _BN * K)], idxb[r], seminb[r])
        pltpu.async_copy(wexp_hbm.at[pl.ds(eb, _BN * K)], wexpb[r],
                         seminb[r])
        pltpu.async_copy(p_hbm.at[pl.ds(nb, _BN)], pb[r], seminb[r])

    def wait_in(r):
        pltpu.make_async_copy(idx_hbm.at[pl.ds(0, _BN * K)], idxb[r],
                              seminb[r]).wait()
        pltpu.make_async_copy(wexp_hbm.at[pl.ds(0, _BN * K)], wexpb[r],
                              seminb[r]).wait()
        pltpu.make_async_copy(p_hbm.at[pl.ds(0, _BN)], pb[r],
                              seminb[r]).wait()

    def fire_gather(r, q):
        pltpu.async_copy(b_hbm.at[idxb[r]], rowsb[q], semgb[q])

    def wait_gather(r, q):
        pltpu.make_async_copy(b_hbm.at[idxb[r]], rowsb[q],
                              semgb[q]).wait()

    def fire_out(bi, q):
        nb = node0 + bi * _BN
        pltpu.async_copy(accb[q], out_hbm.at[pl.ds(nb, _BN)], semoutb[q])

    def wait_out(q):
        pltpu.make_async_copy(accb[q], out_hbm.at[pl.ds(0, _BN)],
                              semoutb[q]).wait()

    def compute(r, q):
        wexp_v, p_v, rows_v, acc_v = wexpb[r], pb[r], rowsb[q], accb[q]
        himask = jnp.full((_LW,), -65536, jnp.int32)

        def node(n, carry):
            e0 = n * K
            ws = [wexp_v[e0 + k] for k in range(K)]
            for cc in range(C // _L):
                plo = p_v[n, pl.ds(cc * _L, _LW)]
                phi = p_v[n, pl.ds(cc * _L + _LW, _LW)]
                alo = jnp.zeros((_LW,), jnp.float32)
                ahi = jnp.zeros((_LW,), jnp.float32)
                for k in range(K):
                    rr = rows_v[e0 + k, pl.ds(cc * _LW, _LW)]
                    rlo = lax.bitcast_convert_type(rr << 16, jnp.float32)
                    rhi = lax.bitcast_convert_type(rr & himask,
                                                   jnp.float32)
                    alo = alo + ws[k] * jnp.maximum(plo + rlo, 0.0)
                    ahi = ahi + ws[k] * jnp.maximum(phi + rhi, 0.0)
                acc_v[n, pl.ds(cc * _L, _LW)] = alo
                acc_v[n, pl.ds(cc * _L + _LW, _LW)] = ahi
            return carry

        lax.fori_loop(0, _BN, node, 0)

    for r in range(4):
        fire_in(r, r)
    wait_in(0)
    fire_gather(0, 0)

    def step(t, carry):
        for u in range(4):
            bi = 4 * t + u
            q = u % 2
            r = u

            @pl.when(bi + 1 < _NB)
            def _():
                wait_in((r + 1) % 4)
                fire_gather((r + 1) % 4, 1 - q)

            wait_gather(r, q)

            @pl.when(bi >= 2)
            def _():
                wait_out(q)

            compute(r, q)
            fire_out(bi, q)

            @pl.when(bi + 4 < _NB)
            def _():
                fire_in(bi + 4, r)
        return carry

    lax.fori_loop(0, _NB // 4, step, 0)
    wait_out(0)
    wait_out(1)


@functools.cache
def _sc_kernel_built():
    return functools.partial(
        pl.kernel,
        out_type=jax.ShapeDtypeStruct((N, C), jnp.float32),
        mesh=plsc.VectorSubcoreMesh(
            core_axis_name="c", subcore_axis_name="s",
            num_cores=_NC, num_subcores=_NS),
        scratch_types=(
            [pltpu.VMEM((_BN * K,), jnp.int32)] * 4
            + [pltpu.VMEM((_BN * K, _LW), jnp.float32)] * 4
            + [pltpu.VMEM((_BN, C), jnp.float32)] * 4
            + [pltpu.VMEM((_BN * K, _CW), jnp.int32)] * 2
            + [pltpu.VMEM((_BN, C), jnp.float32)] * 2
            + [pltpu.SemaphoreType.DMA] * 8
        ),
    )(_sc_body)


def _sc_kernel(p, b, idx_flat, wexp):
    return _sc_kernel_built()(p, b, idx_flat, wexp)


_BLK2 = 1024


def _tc2_body(x_ref, hsum_ref, mW2_ref, mb2_ref,
              uW1_ref, ub1_ref, uW2_ref, ub2_ref, out_ref):
    hagg = (jnp.dot(hsum_ref[...].astype(jnp.float32), mW2_ref[...],
                    preferred_element_type=jnp.float32) + mb2_ref[...])
    xb = x_ref[...]
    u = (jnp.dot(xb, uW1_ref[0:C, :], preferred_element_type=jnp.float32)
         + jnp.dot(hagg, uW1_ref[C:2 * C, :],
                   preferred_element_type=jnp.float32)
         + ub1_ref[...])
    u = jnp.maximum(u, 0.0)
    out_ref[...] = (xb + jnp.dot(u, uW2_ref[...],
                                 preferred_element_type=jnp.float32)
                    + ub2_ref[...])


def _run_tc2(x, hsum, mW2, mb2, uW1, ub1, uW2, ub2):
    nb = N // _BLK2
    return pl.pallas_call(
        _tc2_body,
        grid=(nb,),
        in_specs=[
            pl.BlockSpec((_BLK2, C), lambda i: (i, 0)),
            pl.BlockSpec((_BLK2, C), lambda i: (i, 0)),
            pl.BlockSpec((C, C), lambda i: (0, 0)),
            pl.BlockSpec((1, C), lambda i: (0, 0)),
            pl.BlockSpec((2 * C, C), lambda i: (0, 0)),
            pl.BlockSpec((1, C), lambda i: (0, 0)),
            pl.BlockSpec((C, C), lambda i: (0, 0)),
            pl.BlockSpec((1, C), lambda i: (0, 0)),
        ],
        out_specs=pl.BlockSpec((_BLK2, C), lambda i: (i, 0)),
        out_shape=jax.ShapeDtypeStruct((N, C), jnp.float32),
    )(x, hsum, mW2, mb2, uW1, ub1, uW2, ub2)


def kernel(x, msg_W1, msg_b1, msg_W2, msg_b2, upd_W1, upd_b1, upd_W2, upd_b2):
    w1a = msg_W1[:C]
    w1b = msg_W1[C:]
    idx, wgt, p, b = _run_tc1(x, w1a, w1b, msg_b1.reshape(1, C))
    idx_flat = idx.reshape(-1)
    wexp = jnp.broadcast_to(wgt.reshape(N * K, 1), (N * K, _LW))
    bperm = b.reshape(N, C // _L, 2, _LW).transpose(0, 1, 3, 2)
    b32 = lax.bitcast_convert_type(bperm, jnp.int32).reshape(N, _CW)
    hsum = _sc_kernel(p, b32, idx_flat, wexp)
    return _run_tc2(x, hsum, msg_W2, msg_b2.reshape(1, C),
                    upd_W1, upd_b1.reshape(1, C),
                    upd_W2, upd_b2.reshape(1, C))


# speedup vs baseline: 15.6551x; 1.1585x over previous
import functools

import jax
import jax.numpy as jnp
from jax import lax
from jax.experimental import pallas as pl
from jax.experimental.pallas import tpu as pltpu
from jax.experimental.pallas import tpu_sc as plsc

N = 4096
C = 256
K = 20

BLK = 256
NBLK = N // BLK


def _tc1_body(x_ref, w1a_ref, w1b_ref, b1_ref,
              idx_ref, w_ref, p_ref, b_ref, xn_ref):
    i = pl.program_id(0)

    @pl.when(i == 0)
    def _():
        xf = x_ref[...]
        nrm = jnp.maximum(
            jnp.sqrt(jnp.sum(xf * xf, axis=1, keepdims=True)), 1e-8)
        xn_ref[...] = xf / nrm

    xblk = x_ref[pl.ds(i * BLK, BLK), :]
    p_ref[...] = (jnp.dot(xblk, w1a_ref[...],
                          preferred_element_type=jnp.float32) + b1_ref[...])
    b_ref[...] = jnp.dot(xblk, w1b_ref[...],
                         preferred_element_type=jnp.float32
                         ).astype(jnp.bfloat16)

    xnblk = xn_ref[pl.ds(i * BLK, BLK), :]
    sim = lax.dot_general(xnblk, xn_ref[...],
                          (((1,), (1,)), ((), ())),
                          preferred_element_type=jnp.float32)

    iota = lax.broadcasted_iota(jnp.int32, (BLK, N), 1)
    bits = lax.bitcast_convert_type(sim, jnp.int32)
    skey = jnp.where(bits >= 0, bits, bits ^ jnp.int32(0x7FFFFFFF))
    kk = (skey & jnp.int32(~0xFFF)) | (jnp.int32(N - 1) - iota)

    F = 8
    W = N // F
    Ls = [kk[:, i * W:(i + 1) * W] for i in range(F)]
    _CES = [(0, 1), (2, 3), (4, 5), (6, 7),
            (0, 2), (1, 3), (4, 6), (5, 7),
            (1, 2), (5, 6),
            (0, 4), (1, 5), (2, 6), (3, 7),
            (2, 4), (3, 5),
            (1, 2), (3, 4), (5, 6)]
    for a, c in _CES:
        hi = jnp.maximum(Ls[a], Ls[c])
        lo = jnp.minimum(Ls[a], Ls[c])
        Ls[a], Ls[c] = hi, lo

    MINK = jnp.int32(-2147483648)
    picked = []
    for _ in range(K):
        m = jnp.max(Ls[0], axis=1, keepdims=True)
        picked.append(m)
        eq = Ls[0] == m
        for i in range(F - 1):
            Ls[i] = jnp.where(eq, Ls[i + 1], Ls[i])
        Ls[F - 1] = jnp.where(eq, MINK, Ls[F - 1])

    kcat = jnp.concatenate(picked, axis=1)
    ix = jnp.int32(N - 1) - (kcat & jnp.int32(0xFFF))
    t = kcat & jnp.int32(~0xFFF)
    vbits = jnp.where(t >= 0, t, t ^ jnp.int32(0x7FFFFFFF))
    v = lax.bitcast_convert_type(vbits, jnp.float32)
    e = jnp.exp(v - v[:, 0:1])
    wgt = e / jnp.sum(e, axis=1, keepdims=True)
    idx_ref[...] = ix
    w_ref[...] = wgt


def _run_tc1(x, w1a, w1b, b1):
    return pl.pallas_call(
        _tc1_body,
        grid=(NBLK,),
        in_specs=[
            pl.BlockSpec((N, C), lambda i: (0, 0)),
            pl.BlockSpec((C, C), lambda i: (0, 0)),
            pl.BlockSpec((C, C), lambda i: (0, 0)),
            pl.BlockSpec((1, C), lambda i: (0, 0)),
        ],
        out_specs=[
            pl.BlockSpec((BLK, K), lambda i: (i, 0)),
            pl.BlockSpec((BLK, K), lambda i: (i, 0)),
            pl.BlockSpec((BLK, C), lambda i: (i, 0)),
            pl.BlockSpec((BLK, C), lambda i: (i, 0)),
        ],
        out_shape=[
            jax.ShapeDtypeStruct((N, K), jnp.int32),
            jax.ShapeDtypeStruct((N, K), jnp.float32),
            jax.ShapeDtypeStruct((N, C), jnp.float32),
            jax.ShapeDtypeStruct((N, C), jnp.bfloat16),
        ],
        scratch_shapes=[pltpu.VMEM((N, C), jnp.float32)],
    )(x, w1a, w1b, b1)


_NC = 2
_NS = 16
_NW = _NC * _NS
_NPW = N // _NW
_BN = 4
_NB = _NPW // _BN
_L = 32
_LW = _L // 2
_CW = C // 2


def _sc_body(p_hbm, b_hbm, idx_hbm, wexp_hbm, out_hbm,
             idx0, idx1, idx2, idx3,
             wexp0, wexp1, wexp2, wexp3,
             p0, p1, p2, p3,
             rows0, rows1, acc0, acc1,
             semin0, semin1, semin2, semin3,
             semg0, semg1, semout0, semout1):
    cid = lax.axis_index("c")
    sid = lax.axis_index("s")
    wid = sid * _NC + cid
    node0 = wid * _NPW

    idxb = (idx0, idx1, idx2, idx3)
    wexpb = (wexp0, wexp1, wexp2, wexp3)
    pb = (p0, p1, p2, p3)
    seminb = (semin0, semin1, semin2, semin3)
    rowsb = (rows0, rows1)
    semgb = (semg0, semg1)
    accb = (acc0, acc1)
    semoutb = (semout0, semout1)

    def fire_in(bi, r):
        nb = node0 + bi * _BN
        eb = nb * K
        pltpu.async_copy(idx_hbm.at[pl.ds(eb, _BN * K)], idxb[r], seminb[r])
        pltpu.async_copy(wexp_hbm.at[pl.ds(eb, _BN * K)], wexpb[r],
                         seminb[r])
        pltpu.async_copy(p_hbm.at[pl.ds(nb, _BN)], pb[r], seminb[r])

    def wait_in(r):
        pltpu.make_async_copy(idx_hbm.at[pl.ds(0, _BN * K)], idxb[r],
                              seminb[r]).wait()
        pltpu.make_async_copy(wexp_hbm.at[pl.ds(0, _BN * K)], wexpb[r],
                              seminb[r]).wait()
        pltpu.make_async_copy(p_hbm.at[pl.ds(0, _BN)], pb[r],
                              seminb[r]).wait()

    def fire_gather(r, q):
        pltpu.async_copy(b_hbm.at[idxb[r]], rowsb[q], semgb[q])

    def wait_gather(r, q):
        pltpu.make_async_copy(b_hbm.at[idxb[r]], rowsb[q],
                              semgb[q]).wait()

    def fire_out(bi, q):
        nb = node0 + bi * _BN
        pltpu.async_copy(accb[q], out_hbm.at[pl.ds(nb, _BN)], semoutb[q])

    def wait_out(q):
        pltpu.make_async_copy(accb[q], out_hbm.at[pl.ds(0, _BN)],
                              semoutb[q]).wait()

    def compute(r, q):
        wexp_v, p_v, rows_v, acc_v = wexpb[r], pb[r], rowsb[q], accb[q]
        himask = jnp.full((_LW,), -65536, jnp.int32)

        def node(n, carry):
            e0 = n * K
            ws = [wexp_v[e0 + k] for k in range(K)]
            for cc in range(C // _L):
                plo = p_v[n, pl.ds(cc * _L, _LW)]
                phi = p_v[n, pl.ds(cc * _L + _LW, _LW)]
                alo = jnp.zeros((_LW,), jnp.float32)
                ahi = jnp.zeros((_LW,), jnp.float32)
                for k in range(K):
                    rr = rows_v[e0 + k, pl.ds(cc * _LW, _LW)]
                    rlo = lax.bitcast_convert_type(rr << 16, jnp.float32)
                    rhi = lax.bitcast_convert_type(rr & himask,
                                                   jnp.float32)
                    alo = alo + ws[k] * jnp.maximum(plo + rlo, 0.0)
                    ahi = ahi + ws[k] * jnp.maximum(phi + rhi, 0.0)
                acc_v[n, pl.ds(cc * _L, _LW)] = alo
                acc_v[n, pl.ds(cc * _L + _LW, _LW)] = ahi
            return carry

        lax.fori_loop(0, _BN, node, 0)

    for r in range(4):
        fire_in(r, r)
    wait_in(0)
    fire_gather(0, 0)

    def step(t, carry):
        for u in range(4):
            bi = 4 * t + u
            q = u % 2
            r = u

            @pl.when(bi + 1 < _NB)
            def _():
                wait_in((r + 1) % 4)
                fire_gather((r + 1) % 4, 1 - q)

            wait_gather(r, q)

            @pl.when(bi >= 2)
            def _():
                wait_out(q)

            compute(r, q)
            fire_out(bi, q)

            @pl.when(bi + 4 < _NB)
            def _():
                fire_in(bi + 4, r)
        return carry

    lax.fori_loop(0, _NB // 4, step, 0)
    wait_out(0)
    wait_out(1)


@functools.cache
def _sc_kernel_built():
    return functools.partial(
        pl.kernel,
        out_type=jax.ShapeDtypeStruct((N, C), jnp.float32),
        mesh=plsc.VectorSubcoreMesh(
            core_axis_name="c", subcore_axis_name="s",
            num_cores=_NC, num_subcores=_NS),
        scratch_types=(
            [pltpu.VMEM((_BN * K,), jnp.int32)] * 4
            + [pltpu.VMEM((_BN * K, _LW), jnp.float32)] * 4
            + [pltpu.VMEM((_BN, C), jnp.float32)] * 4
            + [pltpu.VMEM((_BN * K, _CW), jnp.int32)] * 2
            + [pltpu.VMEM((_BN, C), jnp.float32)] * 2
            + [pltpu.SemaphoreType.DMA] * 8
        ),
    )(_sc_body)


def _sc_kernel(p, b, idx_flat, wexp):
    return _sc_kernel_built()(p, b, idx_flat, wexp)


_BLK2 = 1024


def _tc2_body(x_ref, hsum_ref, mW2_ref, mb2_ref,
              uW1_ref, ub1_ref, uW2_ref, ub2_ref, out_ref):
    hagg = (jnp.dot(hsum_ref[...].astype(jnp.float32), mW2_ref[...],
                    preferred_element_type=jnp.float32) + mb2_ref[...])
    xb = x_ref[...]
    u = (jnp.dot(xb, uW1_ref[0:C, :], preferred_element_type=jnp.float32)
         + jnp.dot(hagg, uW1_ref[C:2 * C, :],
                   preferred_element_type=jnp.float32)
         + ub1_ref[...])
    u = jnp.maximum(u, 0.0)
    out_ref[...] = (xb + jnp.dot(u, uW2_ref[...],
                                 preferred_element_type=jnp.float32)
                    + ub2_ref[...])


def _run_tc2(x, hsum, mW2, mb2, uW1, ub1, uW2, ub2):
    nb = N // _BLK2
    return pl.pallas_call(
        _tc2_body,
        grid=(nb,),
        in_specs=[
            pl.BlockSpec((_BLK2, C), lambda i: (i, 0)),
            pl.BlockSpec((_BLK2, C), lambda i: (i, 0)),
            pl.BlockSpec((C, C), lambda i: (0, 0)),
            pl.BlockSpec((1, C), lambda i: (0, 0)),
            pl.BlockSpec((2 * C, C), lambda i: (0, 0)),
            pl.BlockSpec((1, C), lambda i: (0, 0)),
            pl.BlockSpec((C, C), lambda i: (0, 0)),
            pl.BlockSpec((1, C), lambda i: (0, 0)),
        ],
        out_specs=pl.BlockSpec((_BLK2, C), lambda i: (i, 0)),
        out_shape=jax.ShapeDtypeStruct((N, C), jnp.float32),
    )(x, hsum, mW2, mb2, uW1, ub1, uW2, ub2)


def kernel(x, msg_W1, msg_b1, msg_W2, msg_b2, upd_W1, upd_b1, upd_W2, upd_b2):
    w1a = msg_W1[:C]
    w1b = msg_W1[C:]
    idx, wgt, p, b = _run_tc1(x, w1a, w1b, msg_b1.reshape(1, C))
    idx_flat = idx.reshape(-1)
    wexp = jnp.broadcast_to(wgt.reshape(N * K, 1), (N * K, _LW))
    bperm = b.reshape(N, C // _L, 2, _LW).transpose(0, 1, 3, 2)
    b32 = lax.bitcast_convert_type(bperm, jnp.int32).reshape(N, _CW)
    hsum = _sc_kernel(p, b32, idx_flat, wexp)
    return _run_tc2(x, hsum, msg_W2, msg_b2.reshape(1, C),
                    upd_W1, upd_b1.reshape(1, C),
                    upd_W2, upd_b2.reshape(1, C))


# speedup vs baseline: 15.6971x; 1.0027x over previous
import functools

import jax
import jax.numpy as jnp
from jax import lax
from jax.experimental import pallas as pl
from jax.experimental.pallas import tpu as pltpu
from jax.experimental.pallas import tpu_sc as plsc

N = 4096
C = 256
K = 20

BLK = 256
NBLK = N // BLK


GN = 1024
NG = N // GN


def _tc0_body(x_ref, w1a_ref, w1b_ref, b1_ref, xn_ref, p_ref, b_ref):
    xf = x_ref[...]
    nrm = jnp.maximum(
        jnp.sqrt(jnp.sum(xf * xf, axis=1, keepdims=True)), 1e-8)
    xn_ref[...] = xf / nrm
    p_ref[...] = (jnp.dot(xf, w1a_ref[...],
                          preferred_element_type=jnp.float32) + b1_ref[...])
    b_ref[...] = jnp.dot(xf, w1b_ref[...],
                         preferred_element_type=jnp.float32
                         ).astype(jnp.bfloat16)


def _run_tc0(x, w1a, w1b, b1):
    return pl.pallas_call(
        _tc0_body,
        grid=(NG,),
        in_specs=[
            pl.BlockSpec((GN, C), lambda i: (i, 0)),
            pl.BlockSpec((C, C), lambda i: (0, 0)),
            pl.BlockSpec((C, C), lambda i: (0, 0)),
            pl.BlockSpec((1, C), lambda i: (0, 0)),
        ],
        out_specs=[
            pl.BlockSpec((GN, C), lambda i: (i, 0)),
            pl.BlockSpec((GN, C), lambda i: (i, 0)),
            pl.BlockSpec((GN, C), lambda i: (i, 0)),
        ],
        out_shape=[
            jax.ShapeDtypeStruct((N, C), jnp.float32),
            jax.ShapeDtypeStruct((N, C), jnp.float32),
            jax.ShapeDtypeStruct((N, C), jnp.bfloat16),
        ],
    )(x, w1a, w1b, b1)


def _tc1_body(gbase_ref, xn_ref, idx_ref, w_ref):
    i = pl.program_id(0)
    row0 = pl.multiple_of(gbase_ref[0] + i * BLK, BLK)
    xnblk = xn_ref[pl.ds(row0, BLK), :]
    sim = lax.dot_general(xnblk, xn_ref[...],
                          (((1,), (1,)), ((), ())),
                          preferred_element_type=jnp.float32)

    iota = lax.broadcasted_iota(jnp.int32, (BLK, N), 1)
    bits = lax.bitcast_convert_type(sim, jnp.int32)
    skey = jnp.where(bits >= 0, bits, bits ^ jnp.int32(0x7FFFFFFF))
    kk = (skey & jnp.int32(~0xFFF)) | (jnp.int32(N - 1) - iota)

    F = 8
    W = N // F
    Ls = [kk[:, i2 * W:(i2 + 1) * W] for i2 in range(F)]
    _CES = [(0, 1), (2, 3), (4, 5), (6, 7),
            (0, 2), (1, 3), (4, 6), (5, 7),
            (1, 2), (5, 6),
            (0, 4), (1, 5), (2, 6), (3, 7),
            (2, 4), (3, 5),
            (1, 2), (3, 4), (5, 6)]
    for a, c in _CES:
        hi = jnp.maximum(Ls[a], Ls[c])
        lo = jnp.minimum(Ls[a], Ls[c])
        Ls[a], Ls[c] = hi, lo

    MINK = jnp.int32(-2147483648)
    picked = []
    for _ in range(K):
        m = jnp.max(Ls[0], axis=1, keepdims=True)
        picked.append(m)
        eq = Ls[0] == m
        for i2 in range(F - 1):
            Ls[i2] = jnp.where(eq, Ls[i2 + 1], Ls[i2])
        Ls[F - 1] = jnp.where(eq, MINK, Ls[F - 1])

    kcat = jnp.concatenate(picked, axis=1)
    ix = jnp.int32(N - 1) - (kcat & jnp.int32(0xFFF))
    t = kcat & jnp.int32(~0xFFF)
    vbits = jnp.where(t >= 0, t, t ^ jnp.int32(0x7FFFFFFF))
    v = lax.bitcast_convert_type(vbits, jnp.float32)
    e = jnp.exp(v - v[:, 0:1])
    wgt = e / jnp.sum(e, axis=1, keepdims=True)
    idx_ref[...] = ix
    w_ref[...] = wgt


def _run_tc1(gbase, xn):
    return pl.pallas_call(
        _tc1_body,
        grid=(GN // BLK,),
        in_specs=[
            pl.BlockSpec(memory_space=pltpu.SMEM),
            pl.BlockSpec((N, C), lambda i: (0, 0)),
        ],
        out_specs=[
            pl.BlockSpec((BLK, K), lambda i: (i, 0)),
            pl.BlockSpec((BLK, K), lambda i: (i, 0)),
        ],
        out_shape=[
            jax.ShapeDtypeStruct((GN, K), jnp.int32),
            jax.ShapeDtypeStruct((GN, K), jnp.float32),
        ],
    )(gbase, xn)


_NC = 2
_NS = 16
_NW = _NC * _NS
_NPW = GN // _NW
_BN = 4
_NB = _NPW // _BN
_L = 32
_LW = _L // 2
_CW = C // 2


def _sc_body(p_hbm, b_hbm, idx_hbm, wexp_hbm, out_hbm,
             idx0, idx1, idx2, idx3,
             wexp0, wexp1, wexp2, wexp3,
             p0, p1, p2, p3,
             rows0, rows1, acc0, acc1,
             semin0, semin1, semin2, semin3,
             semg0, semg1, semout0, semout1):
    cid = lax.axis_index("c")
    sid = lax.axis_index("s")
    wid = sid * _NC + cid
    node0 = wid * _NPW

    idxb = (idx0, idx1, idx2, idx3)
    wexpb = (wexp0, wexp1, wexp2, wexp3)
    pb = (p0, p1, p2, p3)
    seminb = (semin0, semin1, semin2, semin3)
    rowsb = (rows0, rows1)
    semgb = (semg0, semg1)
    accb = (acc0, acc1)
    semoutb = (semout0, semout1)

    def fire_in(bi, r):
        nb = node0 + bi * _BN
        eb = nb * K
        pltpu.async_copy(idx_hbm.at[pl.ds(eb, _BN * K)], idxb[r], seminb[r])
        pltpu.async_copy(wexp_hbm.at[pl.ds(eb, _BN * K)], wexpb[r],
                         seminb[r])
        pltpu.async_copy(p_hbm.at[pl.ds(nb, _BN)], pb[r], seminb[r])

    def wait_in(r):
        pltpu.make_async_copy(idx_hbm.at[pl.ds(0, _BN * K)], idxb[r],
                              seminb[r]).wait()
        pltpu.make_async_copy(wexp_hbm.at[pl.ds(0, _BN * K)], wexpb[r],
                              seminb[r]).wait()
        pltpu.make_async_copy(p_hbm.at[pl.ds(0, _BN)], pb[r],
                              seminb[r]).wait()

    def fire_gather(r, q):
        pltpu.async_copy(b_hbm.at[idxb[r]], rowsb[q], semgb[q])

    def wait_gather(r, q):
        pltpu.make_async_copy(b_hbm.at[idxb[r]], rowsb[q],
                              semgb[q]).wait()

    def fire_out(bi, q):
        nb = node0 + bi * _BN
        pltpu.async_copy(accb[q], out_hbm.at[pl.ds(nb, _BN)], semoutb[q])

    def wait_out(q):
        pltpu.make_async_copy(accb[q], out_hbm.at[pl.ds(0, _BN)],
                              semoutb[q]).wait()

    def compute(r, q):
        wexp_v, p_v, rows_v, acc_v = wexpb[r], pb[r], rowsb[q], accb[q]
        himask = jnp.full((_LW,), -65536, jnp.int32)

        def node(n, carry):
            e0 = n * K
            ws = [wexp_v[e0 + k] for k in range(K)]
            for cc in range(C // _L):
                plo = p_v[n, pl.ds(cc * _L, _LW)]
                phi = p_v[n, pl.ds(cc * _L + _LW, _LW)]
                alo = jnp.zeros((_LW,), jnp.float32)
                ahi = jnp.zeros((_LW,), jnp.float32)
                for k in range(K):
                    rr = rows_v[e0 + k, pl.ds(cc * _LW, _LW)]
                    rlo = lax.bitcast_convert_type(rr << 16, jnp.float32)
                    rhi = lax.bitcast_convert_type(rr & himask,
                                                   jnp.float32)
                    alo = alo + ws[k] * jnp.maximum(plo + rlo, 0.0)
                    ahi = ahi + ws[k] * jnp.maximum(phi + rhi, 0.0)
                acc_v[n, pl.ds(cc * _L, _LW)] = alo
                acc_v[n, pl.ds(cc * _L + _LW, _LW)] = ahi
            return carry

        lax.fori_loop(0, _BN, node, 0)

    for r in range(4):
        fire_in(r, r)
    wait_in(0)
    fire_gather(0, 0)

    def step(t, carry):
        for u in range(4):
            bi = 4 * t + u
            q = u % 2
            r = u

            @pl.when(bi + 1 < _NB)
            def _():
                wait_in((r + 1) % 4)
                fire_gather((r + 1) % 4, 1 - q)

            wait_gather(r, q)

            @pl.when(bi >= 2)
            def _():
                wait_out(q)

            compute(r, q)
            fire_out(bi, q)

            @pl.when(bi + 4 < _NB)
            def _():
                fire_in(bi + 4, r)
        return carry

    lax.fori_loop(0, _NB // 4, step, 0)
    wait_out(0)
    wait_out(1)


@functools.cache
def _sc_kernel_built():
    return functools.partial(
        pl.kernel,
        out_type=jax.ShapeDtypeStruct((GN, C), jnp.float32),
        mesh=plsc.VectorSubcoreMesh(
            core_axis_name="c", subcore_axis_name="s",
            num_cores=_NC, num_subcores=_NS),
        scratch_types=(
            [pltpu.VMEM((_BN * K,), jnp.int32)] * 4
            + [pltpu.VMEM((_BN * K, _LW), jnp.float32)] * 4
            + [pltpu.VMEM((_BN, C), jnp.float32)] * 4
            + [pltpu.VMEM((_BN * K, _CW), jnp.int32)] * 2
            + [pltpu.VMEM((_BN, C), jnp.float32)] * 2
            + [pltpu.SemaphoreType.DMA] * 8
        ),
    )(_sc_body)


def _sc_kernel(p, b, idx_flat, wexp):
    return _sc_kernel_built()(p, b, idx_flat, wexp)


_BLK2 = 1024


def _tc2_body(x_ref, hsum_ref, mW2_ref, mb2_ref,
              uW1_ref, ub1_ref, uW2_ref, ub2_ref, out_ref):
    hagg = (jnp.dot(hsum_ref[...].astype(jnp.float32), mW2_ref[...],
                    preferred_element_type=jnp.float32) + mb2_ref[...])
    xb = x_ref[...]
    u = (jnp.dot(xb, uW1_ref[0:C, :], preferred_element_type=jnp.float32)
         + jnp.dot(hagg, uW1_ref[C:2 * C, :],
                   preferred_element_type=jnp.float32)
         + ub1_ref[...])
    u = jnp.maximum(u, 0.0)
    out_ref[...] = (xb + jnp.dot(u, uW2_ref[...],
                                 preferred_element_type=jnp.float32)
                    + ub2_ref[...])


def _run_tc2(x, hsum, mW2, mb2, uW1, ub1, uW2, ub2):
    nb = N // _BLK2
    return pl.pallas_call(
        _tc2_body,
        grid=(nb,),
        in_specs=[
            pl.BlockSpec((_BLK2, C), lambda i: (i, 0)),
            pl.BlockSpec((_BLK2, C), lambda i: (i, 0)),
            pl.BlockSpec((C, C), lambda i: (0, 0)),
            pl.BlockSpec((1, C), lambda i: (0, 0)),
            pl.BlockSpec((2 * C, C), lambda i: (0, 0)),
            pl.BlockSpec((1, C), lambda i: (0, 0)),
            pl.BlockSpec((C, C), lambda i: (0, 0)),
            pl.BlockSpec((1, C), lambda i: (0, 0)),
        ],
        out_specs=pl.BlockSpec((_BLK2, C), lambda i: (i, 0)),
        out_shape=jax.ShapeDtypeStruct((N, C), jnp.float32),
    )(x, hsum, mW2, mb2, uW1, ub1, uW2, ub2)


def kernel(x, msg_W1, msg_b1, msg_W2, msg_b2, upd_W1, upd_b1, upd_W2, upd_b2):
    w1a = msg_W1[:C]
    w1b = msg_W1[C:]
    xn, p, b = _run_tc0(x, w1a, w1b, msg_b1.reshape(1, C))
    bperm = b.reshape(N, C // _L, 2, _LW).transpose(0, 1, 3, 2)
    b32 = lax.bitcast_convert_type(bperm, jnp.int32).reshape(N, _CW)

    hs = []
    for g in range(NG):
        gbase = jnp.full((1,), g * GN, jnp.int32)
        idx, wgt = _run_tc1(gbase, xn)
        wexp = jnp.broadcast_to(wgt.reshape(GN * K, 1), (GN * K, _LW))
        pg = lax.slice_in_dim(p, g * GN, (g + 1) * GN, axis=0)
        hs.append(_sc_kernel(pg, b32, idx.reshape(-1), wexp))
    hsum = jnp.concatenate(hs, axis=0)
    return _run_tc2(x, hsum, msg_W2, msg_b2.reshape(1, C),
                    upd_W1, upd_b1.reshape(1, C),
                    upd_W2, upd_b2.reshape(1, C))
